# baseline (speedup 1.0000x reference)
import functools

import jax
import jax.numpy as jnp
from jax import lax
from jax.experimental import pallas as pl
from jax.experimental.pallas import tpu as pltpu
from jax.experimental.pallas import tpu_sc as plsc


def _sc_gather(emb, gates_w, idx3d, n_workers):
    cpw = idx3d.shape[1]
    cw = idx3d.shape[2]
    d = emb.shape[1]
    n = n_workers * cpw * cw
    rpw = cpw * cw

    mesh = plsc.VectorSubcoreMesh(core_axis_name="c", subcore_axis_name="s")
    nc = 2

    @functools.partial(
        pl.kernel,
        out_type=(
            jax.ShapeDtypeStruct((n, d), jnp.float32),
            jax.ShapeDtypeStruct((n,), jnp.float32),
        ),
        mesh=mesh,
        scratch_types=[
            pltpu.VMEM((cpw, cw), jnp.int32),
            pltpu.VMEM((rpw, d), jnp.float32),
            pltpu.VMEM((rpw,), jnp.float32),
            pltpu.SemaphoreType.DMA,
        ],
    )
    def sc_kernel(emb_hbm, gates_hbm, idx_hbm, x_out, g_out,
                  idx_v, rows_v, g_v, sem):
        wid = lax.axis_index("s") * nc + lax.axis_index("c")
        pltpu.sync_copy(idx_hbm.at[wid], idx_v)
        copies = []
        for j in range(cpw):
            copies.append(pltpu.async_copy(
                emb_hbm.at[idx_v.at[j]],
                rows_v.at[pl.ds(j * cw, cw)], sem))
            copies.append(pltpu.async_copy(
                gates_hbm.at[idx_v.at[j]],
                g_v.at[pl.ds(j * cw, cw)], sem))
        for c in copies:
            c.wait()
        base_r = wid * rpw
        pltpu.sync_copy(rows_v, x_out.at[pl.ds(base_r, rpw)])
        pltpu.sync_copy(g_v, g_out.at[pl.ds(base_r, rpw)])

    return sc_kernel(emb, gates_w, idx3d)


def _tc_body(block_b, seq, d, k_blocks):
    seg = d // float(k_blocks)
    lc = 1000000000.0
    df = float(d)
    block_t = block_b * seq

    def body(x_ref, g_ref, w_ref, b_ref, o_ref, *maybe_carry_out):
        xb = x_ref[...]
        gv = g_ref[...] * df
        iota = lax.broadcasted_iota(jnp.int32, (block_t, d), 1).astype(jnp.float32)
        frac = (lc * gv - jnp.floor(lc * gv)) / lc
        mask = (iota < gv).astype(jnp.float32) + frac
        count = jnp.minimum(jnp.ceil(gv), df)
        bidx = jnp.minimum(jnp.floor(count / seg), k_blocks - 1)
        xm = xb * mask
        acc = jnp.zeros((block_t, d), jnp.float32)
        for k in range(k_blocks):
            yk = lax.dot_general(
                xm, w_ref[k], (((1,), (1,)), ((), ())),
                preferred_element_type=jnp.float32)
            yk = yk + b_ref[k][None, :]
            sel = (bidx == k).astype(jnp.float32)
            acc = acc + sel * yk
        out_ref = maybe_carry_out[-1] if maybe_carry_out else o_ref
        out_ref[...] = acc.reshape(block_b, seq, d)

    return body


def _tc_compute(x, g, W, b, bsz, seq, block_b, row0, rows, carry=None):
    n, d = x.shape
    k_blocks = W.shape[0]
    block_t = block_b * seq
    b0 = row0 // block_b

    in_specs = [
        pl.BlockSpec((block_t, d), lambda i: (i, 0)),
        pl.BlockSpec((block_t, 1), lambda i: (i, 0)),
        pl.BlockSpec((k_blocks, d, d), lambda i: (0, 0, 0)),
        pl.BlockSpec((k_blocks, d), lambda i: (0, 0)),
    ]
    out_spec = pl.BlockSpec((block_b, seq, d), lambda i: (i + b0, 0, 0))
    args = (x, g.reshape(n, 1), W, b)
    body = _tc_body(block_b, seq, d, k_blocks)
    if carry is None:
        return pl.pallas_call(
            body,
            grid=(rows // block_b,),
            in_specs=in_specs,
            out_specs=out_spec,
            out_shape=jax.ShapeDtypeStruct((bsz, seq, d), jnp.float32),
        )(*args)
    return pl.pallas_call(
        body,
        grid=(rows // block_b,),
        in_specs=in_specs + [pl.BlockSpec(memory_space=pl.ANY)],
        out_specs=out_spec,
        out_shape=jax.ShapeDtypeStruct((bsz, seq, d), jnp.float32),
        input_output_aliases={4: 0},
    )(*args, carry)


def kernel(input, emb, gates_w, W, b):
    bsz, seq = input.shape
    d = emb.shape[1]
    n = bsz * seq
    n_workers = 32
    half = n // 2
    rows_half = bsz // 2
    gates1d = gates_w.reshape(-1)
    idx4d = input.reshape(2, n_workers, half // (n_workers * 64), 64)
    x0, g0 = _sc_gather(emb, gates1d, idx4d[0], n_workers=n_workers)
    x1, g1 = _sc_gather(emb, gates1d, idx4d[1], n_workers=n_workers)
    out0 = _tc_compute(x0, g0, W, b, bsz, seq, 128, 0, rows_half)
    return _tc_compute(x1, g1, W, b, bsz, seq, 128, rows_half, rows_half,
                       carry=out0)


# speedup vs baseline: 1.1000x; 1.1000x over previous
import functools

import jax
import jax.numpy as jnp
from jax import lax
from jax.experimental import pallas as pl
from jax.experimental.pallas import tpu as pltpu
from jax.experimental.pallas import tpu_sc as plsc


def _sc_gather(emb, gates_w, idx3d, n_workers):
    cpw = idx3d.shape[1]
    cw = idx3d.shape[2]
    d = emb.shape[1]
    n = n_workers * cpw * cw
    rpw = cpw * cw

    mesh = plsc.VectorSubcoreMesh(core_axis_name="c", subcore_axis_name="s")
    nc = 2

    @functools.partial(
        pl.kernel,
        out_type=(
            jax.ShapeDtypeStruct((n, d), jnp.float32),
            jax.ShapeDtypeStruct((n,), jnp.float32),
        ),
        mesh=mesh,
        scratch_types=[
            pltpu.VMEM((cpw, cw), jnp.int32),
            pltpu.VMEM((rpw, d), jnp.float32),
            pltpu.VMEM((rpw,), jnp.float32),
            pltpu.SemaphoreType.DMA,
        ],
    )
    def sc_kernel(emb_hbm, gates_hbm, idx_hbm, x_out, g_out,
                  idx_v, rows_v, g_v, sem):
        wid = lax.axis_index("s") * nc + lax.axis_index("c")
        pltpu.sync_copy(idx_hbm.at[wid], idx_v)
        copies = []
        for j in range(cpw):
            copies.append(pltpu.async_copy(
                emb_hbm.at[idx_v.at[j]],
                rows_v.at[pl.ds(j * cw, cw)], sem))
            copies.append(pltpu.async_copy(
                gates_hbm.at[idx_v.at[j]],
                g_v.at[pl.ds(j * cw, cw)], sem))
        for c in copies:
            c.wait()
        base_r = wid * rpw
        pltpu.sync_copy(rows_v, x_out.at[pl.ds(base_r, rpw)])
        pltpu.sync_copy(g_v, g_out.at[pl.ds(base_r, rpw)])

    return sc_kernel(emb, gates_w, idx3d)


def _tc_body(block_b, seq, d, k_blocks):
    seg = d // float(k_blocks)
    lc = 1000000000.0
    df = float(d)
    block_t = block_b * seq

    def body(x_ref, g_ref, w_ref, b_ref, o_ref, *maybe_carry_out):
        xb = x_ref[...]
        gv = g_ref[...] * df
        iota = lax.broadcasted_iota(jnp.int32, (block_t, d), 1).astype(jnp.float32)
        frac = (lc * gv - jnp.floor(lc * gv)) / lc
        mask = (iota < gv).astype(jnp.float32) + frac
        count = jnp.minimum(jnp.ceil(gv), df)
        bidx = jnp.minimum(jnp.floor(count / seg), k_blocks - 1)
        xm = (xb * mask).astype(jnp.bfloat16)
        acc = jnp.zeros((block_t, d), jnp.float32)
        for k in range(k_blocks):
            yk = lax.dot_general(
                xm, w_ref[k], (((1,), (1,)), ((), ())),
                preferred_element_type=jnp.float32)
            yk = yk + b_ref[k][None, :]
            sel = (bidx == k).astype(jnp.float32)
            acc = acc + sel * yk
        out_ref = maybe_carry_out[-1] if maybe_carry_out else o_ref
        out_ref[...] = acc.reshape(block_b, seq, d)

    return body


def _tc_compute(x, g, W, b, bsz, seq, block_b, row0, rows, carry=None):
    n, d = x.shape
    k_blocks = W.shape[0]
    block_t = block_b * seq
    b0 = row0 // block_b

    in_specs = [
        pl.BlockSpec((block_t, d), lambda i: (i, 0)),
        pl.BlockSpec((block_t, 1), lambda i: (i, 0)),
        pl.BlockSpec((k_blocks, d, d), lambda i: (0, 0, 0)),
        pl.BlockSpec((k_blocks, d), lambda i: (0, 0)),
    ]
    out_spec = pl.BlockSpec((block_b, seq, d), lambda i: (i + b0, 0, 0))
    args = (x, g.reshape(n, 1), W.astype(jnp.bfloat16), b)
    body = _tc_body(block_b, seq, d, k_blocks)
    if carry is None:
        return pl.pallas_call(
            body,
            grid=(rows // block_b,),
            in_specs=in_specs,
            out_specs=out_spec,
            out_shape=jax.ShapeDtypeStruct((bsz, seq, d), jnp.float32),
        )(*args)
    return pl.pallas_call(
        body,
        grid=(rows // block_b,),
        in_specs=in_specs + [pl.BlockSpec(memory_space=pl.ANY)],
        out_specs=out_spec,
        out_shape=jax.ShapeDtypeStruct((bsz, seq, d), jnp.float32),
        input_output_aliases={4: 0},
    )(*args, carry)


def kernel(input, emb, gates_w, W, b):
    bsz, seq = input.shape
    d = emb.shape[1]
    n = bsz * seq
    n_workers = 32
    gates1d = gates_w.reshape(-1)
    idx3d = input.reshape(n_workers, n // (n_workers * 128), 128)
    x, g = _sc_gather(emb, gates1d, idx3d, n_workers=n_workers)
    return _tc_compute(x, g, W, b, bsz, seq, 128, 0, bsz)


# speedup vs baseline: 1.1845x; 1.0768x over previous
import functools

import jax
import jax.numpy as jnp
from jax import lax
from jax.experimental import pallas as pl
from jax.experimental.pallas import tpu as pltpu
from jax.experimental.pallas import tpu_sc as plsc


def _sc_gather(emb, gates_w, idx3d, n_workers):
    cpw = idx3d.shape[1]
    cw = idx3d.shape[2]
    d = emb.shape[1]
    n = n_workers * cpw * cw
    rpw = cpw * cw

    mesh = plsc.VectorSubcoreMesh(core_axis_name="c", subcore_axis_name="s")
    nc = 2

    @functools.partial(
        pl.kernel,
        out_type=(
            jax.ShapeDtypeStruct((n, d), jnp.float32),
            jax.ShapeDtypeStruct((1, n), jnp.float32),
        ),
        mesh=mesh,
        scratch_types=[
            pltpu.VMEM((cpw, cw), jnp.int32),
            pltpu.VMEM((rpw, d), jnp.float32),
            pltpu.VMEM((rpw,), jnp.float32),
            pltpu.SemaphoreType.DMA,
        ],
    )
    def sc_kernel(emb_hbm, gates_hbm, idx_hbm, x_out, g_out,
                  idx_v, rows_v, g_v, sem):
        wid = lax.axis_index("s") * nc + lax.axis_index("c")
        pltpu.sync_copy(idx_hbm.at[wid], idx_v)
        copies = []
        for j in range(cpw):
            copies.append(pltpu.async_copy(
                emb_hbm.at[idx_v.at[j]],
                rows_v.at[pl.ds(j * cw, cw)], sem))
            copies.append(pltpu.async_copy(
                gates_hbm.at[idx_v.at[j]],
                g_v.at[pl.ds(j * cw, cw)], sem))
        for c in copies:
            c.wait()
        base_r = wid * rpw
        pltpu.sync_copy(rows_v, x_out.at[pl.ds(base_r, rpw)])
        pltpu.sync_copy(g_v, g_out.at[0, pl.ds(base_r, rpw)])

    return sc_kernel(emb, gates_w, idx3d)


def _tc_body(block_b, seq, d, k_blocks):
    seg = d // float(k_blocks)
    lc = 1000000000.0
    df = float(d)
    block_t = block_b * seq

    def body(x_ref, g_ref, w_ref, b_ref, o_ref, *maybe_carry_out):
        xb = x_ref[...]
        gv = jnp.transpose(g_ref[...], (1, 0)) * df
        iota = lax.broadcasted_iota(jnp.int32, (block_t, d), 1).astype(jnp.float32)
        frac = (lc * gv - jnp.floor(lc * gv)) / lc
        mask = (iota < gv).astype(jnp.float32) + frac
        count = jnp.minimum(jnp.ceil(gv), df)
        bidx = jnp.minimum(jnp.floor(count / seg), k_blocks - 1)
        xm = (xb * mask).astype(jnp.bfloat16)
        acc = jnp.zeros((block_t, d), jnp.float32)
        for k in range(k_blocks):
            yk = lax.dot_general(
                xm, w_ref[k], (((1,), (1,)), ((), ())),
                preferred_element_type=jnp.float32)
            yk = yk + b_ref[k][None, :]
            sel = (bidx == k).astype(jnp.float32)
            acc = acc + sel * yk
        out_ref = maybe_carry_out[-1] if maybe_carry_out else o_ref
        out_ref[...] = acc.reshape(block_b, seq, d)

    return body


def _tc_compute(x, g, W, b, bsz, seq, block_b, row0, rows, carry=None):
    n, d = x.shape
    k_blocks = W.shape[0]
    block_t = block_b * seq
    b0 = row0 // block_b

    in_specs = [
        pl.BlockSpec((block_t, d), lambda i: (i, 0)),
        pl.BlockSpec((1, block_t), lambda i: (0, i)),
        pl.BlockSpec((k_blocks, d, d), lambda i: (0, 0, 0)),
        pl.BlockSpec((k_blocks, d), lambda i: (0, 0)),
    ]
    out_spec = pl.BlockSpec((block_b, seq, d), lambda i: (i + b0, 0, 0))
    args = (x, g, W.astype(jnp.bfloat16), b)
    body = _tc_body(block_b, seq, d, k_blocks)
    if carry is None:
        return pl.pallas_call(
            body,
            grid=(rows // block_b,),
            in_specs=in_specs,
            out_specs=out_spec,
            out_shape=jax.ShapeDtypeStruct((bsz, seq, d), jnp.float32),
        )(*args)
    return pl.pallas_call(
        body,
        grid=(rows // block_b,),
        in_specs=in_specs + [pl.BlockSpec(memory_space=pl.ANY)],
        out_specs=out_spec,
        out_shape=jax.ShapeDtypeStruct((bsz, seq, d), jnp.float32),
        input_output_aliases={4: 0},
    )(*args, carry)


def kernel(input, emb, gates_w, W, b):
    bsz, seq = input.shape
    d = emb.shape[1]
    n = bsz * seq
    n_workers = 32
    gates1d = gates_w.reshape(-1)
    idx3d = input.reshape(n_workers, n // (n_workers * 128), 128)
    x, g = _sc_gather(emb, gates1d, idx3d, n_workers=n_workers)
    return _tc_compute(x, g, W, b, bsz, seq, 128, 0, bsz)


# speedup vs baseline: 1.1865x; 1.0016x over previous
import functools

import jax
import jax.numpy as jnp
from jax import lax
from jax.experimental import pallas as pl
from jax.experimental.pallas import tpu as pltpu
from jax.experimental.pallas import tpu_sc as plsc


def _sc_gather(emb, gates_w, idx3d, n_workers):
    cpw = idx3d.shape[1]
    cw = idx3d.shape[2]
    d = emb.shape[1]
    n = n_workers * cpw * cw
    rpw = cpw * cw

    mesh = plsc.VectorSubcoreMesh(core_axis_name="c", subcore_axis_name="s")
    nc = 2

    @functools.partial(
        pl.kernel,
        out_type=(
            jax.ShapeDtypeStruct((n, d), jnp.float32),
            jax.ShapeDtypeStruct((1, n), jnp.float32),
        ),
        mesh=mesh,
        scratch_types=[
            pltpu.VMEM((cpw, cw), jnp.int32),
            pltpu.VMEM((rpw, d), jnp.float32),
            pltpu.VMEM((rpw,), jnp.float32),
        ] + [pltpu.SemaphoreType.DMA] * (cpw + 2),
    )
    def sc_kernel(emb_hbm, gates_hbm, idx_hbm, x_out, g_out,
                  idx_v, rows_v, g_v, *sems):
        gsem, wsem = sems[cpw], sems[cpw + 1]
        wid = lax.axis_index("s") * nc + lax.axis_index("c")
        base_r = wid * rpw
        pltpu.sync_copy(idx_hbm.at[wid], idx_v)
        emb_copies = []
        gate_copies = []
        for j in range(cpw):
            emb_copies.append(pltpu.async_copy(
                emb_hbm.at[idx_v.at[j]],
                rows_v.at[pl.ds(j * cw, cw)], sems[j]))
            gate_copies.append(pltpu.async_copy(
                gates_hbm.at[idx_v.at[j]],
                g_v.at[pl.ds(j * cw, cw)], gsem))
        wcopies = []
        for j in range(cpw):
            emb_copies[j].wait()
            wcopies.append(pltpu.async_copy(
                rows_v.at[pl.ds(j * cw, cw)],
                x_out.at[pl.ds(base_r + j * cw, cw)], wsem))
        for c in gate_copies:
            c.wait()
        wcopies.append(pltpu.async_copy(
            g_v, g_out.at[0, pl.ds(base_r, rpw)], wsem))
        for c in wcopies:
            c.wait()

    return sc_kernel(emb, gates_w, idx3d)


def _tc_body(block_b, seq, d, k_blocks):
    seg = d // float(k_blocks)
    lc = 1000000000.0
    df = float(d)
    block_t = block_b * seq

    def body(x_ref, g_ref, w_ref, b_ref, o_ref, *maybe_carry_out):
        xb = x_ref[...]
        gv = jnp.transpose(g_ref[...], (1, 0)) * df
        iota = lax.broadcasted_iota(jnp.int32, (block_t, d), 1).astype(jnp.float32)
        frac = (lc * gv - jnp.floor(lc * gv)) / lc
        mask = (iota < gv).astype(jnp.float32) + frac
        count = jnp.minimum(jnp.ceil(gv), df)
        bidx = jnp.minimum(jnp.floor(count / seg), k_blocks - 1)
        xm = (xb * mask).astype(jnp.bfloat16)
        acc = jnp.zeros((block_t, d), jnp.float32)
        for k in range(k_blocks):
            yk = lax.dot_general(
                xm, w_ref[k], (((1,), (1,)), ((), ())),
                preferred_element_type=jnp.float32)
            yk = yk + b_ref[k][None, :]
            sel = (bidx == k).astype(jnp.float32)
            acc = acc + sel * yk
        out_ref = maybe_carry_out[-1] if maybe_carry_out else o_ref
        out_ref[...] = acc.reshape(block_b, seq, d)

    return body


def _tc_compute(x, g, W, b, bsz, seq, block_b, row0, rows, carry=None):
    n, d = x.shape
    k_blocks = W.shape[0]
    block_t = block_b * seq
    b0 = row0 // block_b

    in_specs = [
        pl.BlockSpec((block_t, d), lambda i: (i, 0)),
        pl.BlockSpec((1, block_t), lambda i: (0, i)),
        pl.BlockSpec((k_blocks, d, d), lambda i: (0, 0, 0)),
        pl.BlockSpec((k_blocks, d), lambda i: (0, 0)),
    ]
    out_spec = pl.BlockSpec((block_b, seq, d), lambda i: (i + b0, 0, 0))
    args = (x, g, W.astype(jnp.bfloat16), b)
    body = _tc_body(block_b, seq, d, k_blocks)
    if carry is None:
        return pl.pallas_call(
            body,
            grid=(rows // block_b,),
            in_specs=in_specs,
            out_specs=out_spec,
            out_shape=jax.ShapeDtypeStruct((bsz, seq, d), jnp.float32),
        )(*args)
    return pl.pallas_call(
        body,
        grid=(rows // block_b,),
        in_specs=in_specs + [pl.BlockSpec(memory_space=pl.ANY)],
        out_specs=out_spec,
        out_shape=jax.ShapeDtypeStruct((bsz, seq, d), jnp.float32),
        input_output_aliases={4: 0},
    )(*args, carry)


def kernel(input, emb, gates_w, W, b):
    bsz, seq = input.shape
    d = emb.shape[1]
    n = bsz * seq
    n_workers = 32
    gates1d = gates_w.reshape(-1)
    idx3d = input.reshape(n_workers, n // (n_workers * 128), 128)
    x, g = _sc_gather(emb, gates1d, idx3d, n_workers=n_workers)
    return _tc_compute(x, g, W, b, bsz, seq, 128, 0, bsz)
